# Initial kernel scaffold; baseline (speedup 1.0000x reference)
import jax
import jax.numpy as jnp
from jax.experimental import pallas as pl


def kernel(inputs, pos_table):
    raise NotImplementedError("write your pallas kernel here")


# speedup vs baseline: 2.1558x; 2.1558x over previous
import jax
import jax.numpy as jnp
from jax.experimental import pallas as pl

_SEQ_BLOCK = 256


def _add_kernel(in_ref, table_ref, out_ref):
    out_ref[...] = in_ref[...] + table_ref[...][None, :, :]


def kernel(inputs, pos_table):
    batch, seq_len, out_dim = inputs.shape
    grid = (seq_len // _SEQ_BLOCK,)
    return pl.pallas_call(
        _add_kernel,
        grid=grid,
        in_specs=[
            pl.BlockSpec((batch, _SEQ_BLOCK, out_dim), lambda i: (0, i, 0)),
            pl.BlockSpec((_SEQ_BLOCK, out_dim), lambda i: (i, 0)),
        ],
        out_specs=pl.BlockSpec((batch, _SEQ_BLOCK, out_dim), lambda i: (0, i, 0)),
        out_shape=jax.ShapeDtypeStruct(inputs.shape, inputs.dtype),
    )(inputs, pos_table)
